# Initial kernel scaffold; baseline (speedup 1.0000x reference)
import functools

import jax
import jax.numpy as jnp
from jax import lax
from jax.experimental import pallas as pl
from jax.experimental.pallas import tpu as pltpu
from jax.experimental.pallas import tpu_sc as plsc

N = 10000
E = 320000
D = 128
A = 64
H = 128
G = 128

NC = 2
NS = 16
NW = NC * NS
CH = 128
NCHUNK = 80
EPT = CH * NCHUNK
EPAD = EPT * NW
NPAD = 10240
RPT = NPAD // NS

_mesh = plsc.VectorSubcoreMesh(core_axis_name="c", subcore_axis_name="s")


@functools.partial(
    pl.kernel,
    out_type=jax.ShapeDtypeStruct((NC, NPAD, 16), jnp.float32),
    mesh=_mesh,
    scratch_types=[
        pltpu.VMEM((NCHUNK, CH), jnp.int32),
        pltpu.VMEM((CH, 16), jnp.float32),
        pltpu.VMEM_SHARED((NPAD, 16), jnp.float32),
    ],
)
def _sc_degree(dst_hbm, ones_hbm, zer_hbm, hist_hbm, dst_v, ones_v, hist_sh):
    c = lax.axis_index("c")
    s = lax.axis_index("s")
    wid = c * NS + s
    pltpu.sync_copy(dst_hbm.at[wid], dst_v)
    pltpu.sync_copy(ones_hbm, ones_v)
    pltpu.sync_copy(zer_hbm, hist_sh.at[pl.ds(s * RPT, RPT)])
    plsc.subcore_barrier()

    @pl.loop(0, NCHUNK)
    def _(j):
        pltpu.sync_copy(ones_v, hist_sh.at[dst_v.at[j]], add=True)

    plsc.subcore_barrier()
    pltpu.sync_copy(hist_sh.at[pl.ds(s * RPT, RPT)],
                    hist_hbm.at[c, pl.ds(s * RPT, RPT)])


@functools.partial(
    pl.kernel,
    out_type=jax.ShapeDtypeStruct((NC, NPAD, H), jnp.float32),
    mesh=_mesh,
    scratch_types=[
        pltpu.VMEM((NCHUNK, CH), jnp.int32),
        pltpu.VMEM((NCHUNK, CH), jnp.int32),
        pltpu.VMEM((CH, H), jnp.float32),
        pltpu.VMEM_SHARED((NPAD, H), jnp.float32),
    ],
)
def _sc_scatter(y_hbm, src_hbm, dst_hbm, zer_hbm, acc_hbm,
                src_v, dst_v, rows_v, acc_sh):
    c = lax.axis_index("c")
    s = lax.axis_index("s")
    wid = c * NS + s
    pltpu.sync_copy(src_hbm.at[wid], src_v)
    pltpu.sync_copy(dst_hbm.at[wid], dst_v)
    pltpu.sync_copy(zer_hbm, acc_sh.at[pl.ds(s * RPT, RPT)])
    plsc.subcore_barrier()

    @pl.loop(0, NCHUNK)
    def _(j):
        pltpu.sync_copy(y_hbm.at[src_v.at[j]], rows_v)
        pltpu.sync_copy(rows_v, acc_sh.at[dst_v.at[j]], add=True)

    plsc.subcore_barrier()
    pltpu.sync_copy(acc_sh.at[pl.ds(s * RPT, RPT)],
                    acc_hbm.at[c, pl.ds(s * RPT, RPT)])


_P = lax.Precision.HIGHEST


def _dotT(a, b):
    return lax.dot_general(a, b, (((1,), (1,)), ((), ())),
                           preferred_element_type=jnp.float32, precision=_P)


def _bn(h, g, b):
    mu = jnp.mean(h, axis=0, keepdims=True)
    var = jnp.mean((h - mu) ** 2, axis=0, keepdims=True)
    return (h - mu) * lax.rsqrt(var + 1e-5) * g + b


def _tc_dense1_body(x_ref, act_ref, wg_ref, bg_ref, w2_ref, b2_ref,
                    g2_ref, be2_ref, w3b_ref, sup_ref, z2_ref):
    sup_ref[...] = _dotT(x_ref[...], wg_ref[...]) + bg_ref[...]
    h2 = jax.nn.relu(_dotT(act_ref[...], w2_ref[...]) + b2_ref[...])
    h2n = _bn(h2, g2_ref[...], be2_ref[...])
    z2_ref[...] = jnp.sum(h2n * w3b_ref[...], axis=1, keepdims=True)


def _tc_scale_body(sup_ref, hist_ref, y_ref, dinv_ref):
    deg = hist_ref[0, :N, 0:1] + hist_ref[1, :N, 0:1] + 1.0
    dinv = lax.rsqrt(deg)
    dinv_ref[...] = dinv
    y_ref[...] = sup_ref[...] * dinv


def _tc_final_body(acc_ref, y_ref, dinv_ref, z2_ref, batch_ref, w1_ref,
                   b1_ref, g0_ref, be0_ref, g1_ref, be1_ref, w3a_ref,
                   b3_ref, out_ref):
    aggr = acc_ref[0, :N, :] + acc_ref[1, :N, :] + y_ref[...]
    h0 = jax.nn.relu(aggr * dinv_ref[...])
    h0n = _bn(h0, g0_ref[...], be0_ref[...])
    h1 = jax.nn.relu(_dotT(h0n, w1_ref[...]) + b1_ref[...])
    h1n = _bn(h1, g1_ref[...], be1_ref[...])
    z = jnp.sum(h1n * w3a_ref[...], axis=1, keepdims=True) + z2_ref[...]
    zc = jnp.concatenate([z, jnp.ones_like(z)], axis=1)
    gid = lax.broadcasted_iota(jnp.int32, (G, N), 0)
    onehot = (batch_ref[...] == gid).astype(jnp.float32)
    sc = lax.dot_general(onehot, zc, (((1,), (0,)), ((), ())),
                         preferred_element_type=jnp.float32, precision=_P)
    out_ref[...] = sc[:, 0:1] / jnp.maximum(sc[:, 1:2], 1.0) + b3_ref[...]


_tc_dense1 = pl.pallas_call(
    _tc_dense1_body,
    out_shape=[jax.ShapeDtypeStruct((N, H), jnp.float32),
               jax.ShapeDtypeStruct((N, 1), jnp.float32)],
)

_tc_scale = pl.pallas_call(
    _tc_scale_body,
    out_shape=[jax.ShapeDtypeStruct((N, H), jnp.float32),
               jax.ShapeDtypeStruct((N, 1), jnp.float32)],
)

_tc_final = pl.pallas_call(
    _tc_final_body,
    out_shape=jax.ShapeDtypeStruct((G, 1), jnp.float32),
)


def kernel(x, edge_index, batch_size, action, W_gcn, b_gcn, bn0_g, bn0_b,
           W1, b1, bn1_g, bn1_b, W2, b2, bn2_g, bn2_b, W3, b3):
    src = edge_index[0]
    dst = edge_index[1]
    pad = EPAD - E
    src_p = jnp.concatenate([src, jnp.zeros((pad,), jnp.int32)])
    dst_p = jnp.concatenate([dst, jnp.full((pad,), N, jnp.int32)])
    src_p = src_p.reshape(NW, NCHUNK, CH)
    dst_p = dst_p.reshape(NW, NCHUNK, CH)

    ones16 = jnp.ones((CH, 16), jnp.float32)
    zer16 = jnp.zeros((RPT, 16), jnp.float32)
    zerH = jnp.zeros((RPT, H), jnp.float32)

    hist = _sc_degree(dst_p, ones16, zer16)

    support, z2 = _tc_dense1(
        x, action, W_gcn, b_gcn.reshape(1, H), W2, b2.reshape(1, H),
        bn2_g.reshape(1, H), bn2_b.reshape(1, H), W3[:, H:2 * H])

    y, dinv = _tc_scale(support, hist)

    acc = _sc_scatter(y, src_p, dst_p, zerH)

    qval = _tc_final(
        acc, y, dinv, z2, batch_size.reshape(1, N), W1, b1.reshape(1, H),
        bn0_g.reshape(1, H), bn0_b.reshape(1, H), bn1_g.reshape(1, H),
        bn1_b.reshape(1, H), W3[:, :H], b3.reshape(1, 1))
    return qval


# speedup vs baseline: 10.6777x; 10.6777x over previous
import dataclasses
import functools

import jax
import jax.numpy as jnp
from jax import lax
from jax.experimental import pallas as pl
from jax.experimental.pallas import tpu as pltpu
from jax.experimental.pallas import tpu_sc as plsc

N = 10000
E = 320000
D = 128
A = 64
H = 128
G = 128

NC = 2
NS = 16
NW = NC * NS
CH = 128
NCHUNK = 80
EPT = CH * NCHUNK
EPAD = EPT * NW
NPAD = 10240
RPT = NPAD // NS


@functools.cache
def _sc_kernels():
    mesh = plsc.VectorSubcoreMesh(core_axis_name="c", subcore_axis_name="s",
                                  num_cores=NC)
    cp = pltpu.CompilerParams()
    if "needs_layout_passes" in pltpu.CompilerParams.__dataclass_fields__:
        cp = dataclasses.replace(cp, needs_layout_passes=False)

    @functools.partial(
        pl.kernel,
        out_type=jax.ShapeDtypeStruct((NW, NPAD), jnp.float32),
        mesh=mesh,
        compiler_params=cp,
        scratch_types=[
            pltpu.VMEM((NCHUNK, CH), jnp.int32),
            pltpu.VMEM((NPAD,), jnp.float32),
        ],
    )
    def sc_degree(dst_hbm, hist_hbm, dst_v, hist_v):
        c = lax.axis_index("c")
        s = lax.axis_index("s")
        wid = c * NS + s
        pltpu.sync_copy(dst_hbm.at[wid], dst_v)

        @pl.loop(0, NPAD // 16)
        def _(i):
            hist_v[pl.ds(i * 16, 16)] = jnp.zeros((16,), jnp.float32)

        ones = jnp.ones((16,), jnp.float32)

        @pl.loop(0, NCHUNK)
        def _(j):
            @pl.loop(0, CH // 16)
            def _(g):
                idx = dst_v[j, pl.ds(g * 16, 16)]
                plsc.addupdate_scatter(hist_v, [idx], ones)

        pltpu.sync_copy(hist_v, hist_hbm.at[wid])

    @functools.partial(
        pl.kernel,
        out_type=jax.ShapeDtypeStruct((NC, NPAD, H), jnp.float32),
        mesh=mesh,
        scratch_types=[
            pltpu.VMEM((NCHUNK, CH), jnp.int32),
            pltpu.VMEM((NCHUNK, CH), jnp.int32),
            pltpu.VMEM((CH, H), jnp.float32),
            pltpu.VMEM_SHARED((NPAD, H), jnp.float32),
        ],
    )
    def sc_scatter(y_hbm, src_hbm, dst_hbm, zer_hbm, acc_hbm,
                   src_v, dst_v, rows_v, acc_sh):
        c = lax.axis_index("c")
        s = lax.axis_index("s")
        wid = c * NS + s
        pltpu.sync_copy(src_hbm.at[wid], src_v)
        pltpu.sync_copy(dst_hbm.at[wid], dst_v)
        pltpu.sync_copy(zer_hbm, acc_sh.at[pl.ds(s * RPT, RPT)])
        plsc.subcore_barrier()

        @pl.loop(0, NCHUNK)
        def _(j):
            pltpu.sync_copy(y_hbm.at[src_v.at[j]], rows_v)
            pltpu.sync_copy(rows_v, acc_sh.at[dst_v.at[j]], add=True)

        plsc.subcore_barrier()
        pltpu.sync_copy(acc_sh.at[pl.ds(s * RPT, RPT)],
                        acc_hbm.at[c, pl.ds(s * RPT, RPT)])

    return sc_degree, sc_scatter


_P = lax.Precision.HIGHEST


def _dotT(a, b):
    return lax.dot_general(a, b, (((1,), (1,)), ((), ())),
                           preferred_element_type=jnp.float32, precision=_P)


def _bn(h, g, b):
    mu = jnp.mean(h, axis=0, keepdims=True)
    var = jnp.mean((h - mu) ** 2, axis=0, keepdims=True)
    return (h - mu) * lax.rsqrt(var + 1e-5) * g + b


def _tc_dense1_body(x_ref, act_ref, wg_ref, bg_ref, w2_ref, b2_ref,
                    g2_ref, be2_ref, w3b_ref, sup_ref, z2_ref):
    sup_ref[...] = _dotT(x_ref[...], wg_ref[...]) + bg_ref[...]
    h2 = jax.nn.relu(_dotT(act_ref[...], w2_ref[...]) + b2_ref[...])
    h2n = _bn(h2, g2_ref[...], be2_ref[...])
    z2_ref[...] = jnp.sum(h2n * w3b_ref[...], axis=1, keepdims=True)


def _tc_scale_body(sup_ref, hist_ref, y_ref, dinv_ref):
    deg = jnp.sum(hist_ref[:, :N], axis=0)[:, None] + 1.0
    dinv = lax.rsqrt(deg)
    dinv_ref[...] = dinv
    y_ref[...] = sup_ref[...] * dinv


def _tc_final_body(acc_ref, y_ref, dinv_ref, z2_ref, batch_ref, w1_ref,
                   b1_ref, g0_ref, be0_ref, g1_ref, be1_ref, w3a_ref,
                   b3_ref, out_ref):
    aggr = acc_ref[0, :N, :] + acc_ref[1, :N, :] + y_ref[...]
    h0 = jax.nn.relu(aggr * dinv_ref[...])
    h0n = _bn(h0, g0_ref[...], be0_ref[...])
    h1 = jax.nn.relu(_dotT(h0n, w1_ref[...]) + b1_ref[...])
    h1n = _bn(h1, g1_ref[...], be1_ref[...])
    z = jnp.sum(h1n * w3a_ref[...], axis=1, keepdims=True) + z2_ref[...]
    zc = jnp.concatenate([z, jnp.ones_like(z)], axis=1)
    gid = lax.broadcasted_iota(jnp.int32, (G, N), 0)
    onehot = (batch_ref[...] == gid).astype(jnp.float32)
    sc = lax.dot_general(onehot, zc, (((1,), (0,)), ((), ())),
                         preferred_element_type=jnp.float32, precision=_P)
    out_ref[...] = sc[:, 0:1] / jnp.maximum(sc[:, 1:2], 1.0) + b3_ref[...]


_tc_dense1 = pl.pallas_call(
    _tc_dense1_body,
    out_shape=[jax.ShapeDtypeStruct((N, H), jnp.float32),
               jax.ShapeDtypeStruct((N, 1), jnp.float32)],
)

_tc_scale = pl.pallas_call(
    _tc_scale_body,
    out_shape=[jax.ShapeDtypeStruct((N, H), jnp.float32),
               jax.ShapeDtypeStruct((N, 1), jnp.float32)],
)

_tc_final = pl.pallas_call(
    _tc_final_body,
    out_shape=jax.ShapeDtypeStruct((G, 1), jnp.float32),
)


def kernel(x, edge_index, batch_size, action, W_gcn, b_gcn, bn0_g, bn0_b,
           W1, b1, bn1_g, bn1_b, W2, b2, bn2_g, bn2_b, W3, b3):
    src = edge_index[0]
    dst = edge_index[1]
    pad = EPAD - E
    src_p = jnp.concatenate([src, jnp.zeros((pad,), jnp.int32)])
    dst_p = jnp.concatenate([dst, jnp.full((pad,), N, jnp.int32)])
    src_p = src_p.reshape(NW, NCHUNK, CH)
    dst_p = dst_p.reshape(NW, NCHUNK, CH)

    zerH = jnp.zeros((RPT, H), jnp.float32)

    sc_degree, sc_scatter = _sc_kernels()
    hist = sc_degree(dst_p)

    support, z2 = _tc_dense1(
        x, action, W_gcn, b_gcn.reshape(1, H), W2, b2.reshape(1, H),
        bn2_g.reshape(1, H), bn2_b.reshape(1, H), W3[:, H:2 * H])

    y, dinv = _tc_scale(support, hist)

    acc = sc_scatter(y, src_p, dst_p, zerH)

    qval = _tc_final(
        acc, y, dinv, z2, batch_size.reshape(1, N), W1, b1.reshape(1, H),
        bn0_g.reshape(1, H), bn0_b.reshape(1, H), bn1_g.reshape(1, H),
        bn1_b.reshape(1, H), W3[:, :H], b3.reshape(1, 1))
    return qval
